# baseline (speedup 1.0000x reference)
import jax
import jax.numpy as jnp
from jax.experimental import pallas as pl
from jax.experimental.pallas import tpu as pltpu

_N = 8192
_BN = 4096
_NI = _N // _BN


def _chamfer_body(pc0_ref, pc1_ref, out_ref, dist1_ref, aug1_ref, s0_ref):
    i = pl.program_id(0)

    @pl.when(i == 0)
    def _init():
        dist1_ref[...] = jnp.full((1, _N), jnp.inf, jnp.float32)
        p1 = pc1_ref[...].T
        sq1 = jnp.sum(p1 * p1, axis=0, keepdims=True)
        h = sq1.astype(jnp.bfloat16).astype(jnp.float32)
        r = sq1 - h
        m = r.astype(jnp.bfloat16).astype(jnp.float32)
        l = (r - m).astype(jnp.bfloat16).astype(jnp.float32)
        aug1_ref[...] = jnp.concatenate([p1, h, m, l], axis=0)
        s0_ref[0] = 0.0

    a = pc0_ref[pl.ds(i * _BN, _BN), :]
    aug0 = jnp.concatenate(
        [-2.0 * a, jnp.ones((_BN, 3), jnp.float32)], axis=1)
    t = jax.lax.dot_general(
        aug0, aug1_ref[...],
        dimension_numbers=(((1,), (0,)), ((), ())),
        preferred_element_type=jnp.float32)
    sq0 = jnp.sum(a * a, axis=1)

    rm = jnp.min(t, axis=1)
    s0_ref[0] += jnp.sum(jnp.maximum(rm + sq0, 0.0))
    cm = jnp.min(t + sq0[:, None], axis=0, keepdims=True)
    dist1_ref[...] = jnp.minimum(dist1_ref[...], cm)

    @pl.when(i == _NI - 1)
    def _fin():
        d1 = jnp.maximum(dist1_ref[...], 0.0)
        out_ref[0] = (s0_ref[0] + jnp.sum(d1)) / _N


def kernel(input0, input1):
    out = pl.pallas_call(
        _chamfer_body,
        grid=(_NI,),
        in_specs=[
            pl.BlockSpec((_N, 3), lambda i: (0, 0)),
            pl.BlockSpec((_N, 3), lambda i: (0, 0)),
        ],
        out_specs=pl.BlockSpec(memory_space=pltpu.SMEM),
        out_shape=jax.ShapeDtypeStruct((1,), jnp.float32),
        scratch_shapes=[
            pltpu.VMEM((1, _N), jnp.float32),
            pltpu.VMEM((6, _N), jnp.float32),
            pltpu.SMEM((1,), jnp.float32),
        ],
    )(input0, input1)
    return out[0]


# speedup vs baseline: 1.0927x; 1.0927x over previous
import jax
import jax.numpy as jnp
from jax.experimental import pallas as pl
from jax.experimental.pallas import tpu as pltpu

_N = 8192
_BN = 2048
_NI = _N // _BN


def _chamfer_body(pc0_ref, pc1t_ref, out_ref, dist1_ref, aug1_ref, s0_ref):
    i = pl.program_id(0)

    @pl.when(i == 0)
    def _init():
        dist1_ref[...] = jnp.full((1, _N), jnp.inf, jnp.float32)
        p1 = pc1t_ref[...]
        sq1 = jnp.sum(p1 * p1, axis=0, keepdims=True)
        h = sq1.astype(jnp.bfloat16).astype(jnp.float32)
        r = sq1 - h
        m = r.astype(jnp.bfloat16).astype(jnp.float32)
        l = (r - m).astype(jnp.bfloat16).astype(jnp.float32)
        aug1_ref[...] = jnp.concatenate([p1, h, m, l], axis=0)
        s0_ref[0] = 0.0

    a = pc0_ref[pl.ds(i * _BN, _BN), :]
    aug0 = jnp.concatenate(
        [-2.0 * a, jnp.ones((_BN, 3), jnp.float32)], axis=1)
    t = jax.lax.dot_general(
        aug0, aug1_ref[...],
        dimension_numbers=(((1,), (0,)), ((), ())),
        preferred_element_type=jnp.float32)
    sq0 = jnp.sum(a * a, axis=1)

    rm = jnp.min(t, axis=1)
    s0_ref[0] += jnp.sum(jnp.maximum(rm + sq0, 0.0))
    cm = jnp.min(t + sq0[:, None], axis=0, keepdims=True)
    dist1_ref[...] = jnp.minimum(dist1_ref[...], cm)

    @pl.when(i == _NI - 1)
    def _fin():
        d1 = jnp.maximum(dist1_ref[...], 0.0)
        out_ref[0] = (s0_ref[0] + jnp.sum(d1)) / _N


def kernel(input0, input1):
    pc1t = input1.T

    out = pl.pallas_call(
        _chamfer_body,
        grid=(_NI,),
        in_specs=[
            pl.BlockSpec((_N, 3), lambda i: (0, 0)),
            pl.BlockSpec((3, _N), lambda i: (0, 0)),
        ],
        out_specs=pl.BlockSpec(memory_space=pltpu.SMEM),
        out_shape=jax.ShapeDtypeStruct((1,), jnp.float32),
        scratch_shapes=[
            pltpu.VMEM((1, _N), jnp.float32),
            pltpu.VMEM((6, _N), jnp.float32),
            pltpu.SMEM((1,), jnp.float32),
        ],
    )(input0, pc1t)
    return out[0]


# speedup vs baseline: 1.1133x; 1.0188x over previous
import jax
import jax.numpy as jnp
from jax.experimental import pallas as pl
from jax.experimental.pallas import tpu as pltpu

_N = 8192
_BN = 4096
_NI = _N // _BN


def _chamfer_body(pc0_ref, pc1t_ref, out_ref, dist1_ref, aug1_ref, s0_ref):
    i = pl.program_id(0)

    @pl.when(i == 0)
    def _init():
        dist1_ref[...] = jnp.full((1, _N), jnp.inf, jnp.float32)
        p1 = pc1t_ref[...]
        sq1 = jnp.sum(p1 * p1, axis=0, keepdims=True)
        h = sq1.astype(jnp.bfloat16).astype(jnp.float32)
        r = sq1 - h
        m = r.astype(jnp.bfloat16).astype(jnp.float32)
        l = (r - m).astype(jnp.bfloat16).astype(jnp.float32)
        aug1_ref[...] = jnp.concatenate([p1, h, m, l], axis=0)
        s0_ref[0] = 0.0

    a = pc0_ref[pl.ds(i * _BN, _BN), :]
    aug0 = jnp.concatenate(
        [-2.0 * a, jnp.ones((_BN, 3), jnp.float32)], axis=1)
    t = jax.lax.dot_general(
        aug0, aug1_ref[...],
        dimension_numbers=(((1,), (0,)), ((), ())),
        preferred_element_type=jnp.float32)
    sq0 = jnp.sum(a * a, axis=1)

    rm = jnp.min(t, axis=1)
    s0_ref[0] += jnp.sum(jnp.maximum(rm + sq0, 0.0))
    cm = jnp.min(t + sq0[:, None], axis=0, keepdims=True)
    dist1_ref[...] = jnp.minimum(dist1_ref[...], cm)

    @pl.when(i == _NI - 1)
    def _fin():
        d1 = jnp.maximum(dist1_ref[...], 0.0)
        out_ref[0] = (s0_ref[0] + jnp.sum(d1)) / _N


def kernel(input0, input1):
    pc1t = input1.T

    out = pl.pallas_call(
        _chamfer_body,
        grid=(_NI,),
        in_specs=[
            pl.BlockSpec((_N, 3), lambda i: (0, 0)),
            pl.BlockSpec((3, _N), lambda i: (0, 0)),
        ],
        out_specs=pl.BlockSpec(memory_space=pltpu.SMEM),
        out_shape=jax.ShapeDtypeStruct((1,), jnp.float32),
        scratch_shapes=[
            pltpu.VMEM((1, _N), jnp.float32),
            pltpu.VMEM((6, _N), jnp.float32),
            pltpu.SMEM((1,), jnp.float32),
        ],
    )(input0, pc1t)
    return out[0]
